# baseline (speedup 1.0000x reference)
import functools

import jax
import jax.numpy as jnp
from jax import lax
from jax.experimental import pallas as pl
from jax.experimental.pallas import tpu as pltpu
from jax.experimental.pallas import tpu_sc as plsc

B = 16384
P = 10
NFRAMES = 20
D = 32
NROWS = 1000
OFFSET = 500
NC = 2
NS = 16
NW = NC * NS
BPW = B // NW

_mesh = plsc.VectorSubcoreMesh(core_axis_name="c", subcore_axis_name="s")


@functools.partial(
    pl.kernel,
    out_type=jax.ShapeDtypeStruct((D, P * B), jnp.float32),
    mesh=_mesh,
    compiler_params=pltpu.CompilerParams(
        needs_layout_passes=False, use_tc_tiling_on_sc=True),
    scratch_types=[
        pltpu.VMEM((NFRAMES * BPW,), jnp.int32),
        pltpu.VMEM((D * NROWS,), jnp.float32),
        pltpu.VMEM((P * BPW,), jnp.int32),
        pltpu.VMEM((D // 8, 8, BPW), jnp.float32),
        pltpu.VMEM((D // 8, 8, BPW), jnp.float32),
        pltpu.SemaphoreType.DMA,
        pltpu.SemaphoreType.DMA,
        pltpu.SemaphoreType.DMA,
    ],
)
def _sc_lookup(fi_hbm, table_hbm, out_hbm, fi_v, tab_v, idx_v,
               buf0, buf1, fsem, ssem0, ssem1):
    wid = lax.axis_index("s") * NC + lax.axis_index("c")
    base_b = wid * BPW

    tcopy = pltpu.async_copy(table_hbm, tab_v, fsem)
    fcopies = [
        pltpu.async_copy(fi_hbm.at[pl.ds(f * B + base_b, BPW)],
                         fi_v.at[pl.ds(f * BPW, BPW)], fsem)
        for f in range(NFRAMES)
    ]
    tcopy.wait()
    for cp in fcopies:
        cp.wait()

    @plsc.parallel_loop(0, P * (BPW // 16))
    def compute_idx(k):
        p = k // (BPW // 16)
        j = k - p * (BPW // 16)
        nxt = fi_v[pl.ds((p + 10) * BPW + j * 16, 16)]
        prv = fi_v[pl.ds(p * BPW + j * 16, 16)]
        idx_v[pl.ds(k * 16, 16)] = nxt - prv + OFFSET

    bufs = (buf0, buf1)
    ssem = (ssem0, ssem1)

    def lookup_block(p, buf):
        @plsc.parallel_loop(0, BPW // 16)
        def group(j0):
            idx = idx_v[pl.ds(p * BPW + j0 * 16, 16)]
            vals = [
                plsc.load_gather(tab_v.at[pl.ds(c * NROWS, NROWS)], [idx])
                for c in range(D)
            ]
            for c in range(D):
                buf[c // 8, c % 8, pl.ds(j0 * 16, 16)] = vals[c]

    scat = [None, None]
    for p in range(P):
        bsel = p & 1
        if scat[bsel] is not None:
            for cp in scat[bsel]:
                cp.wait()
            scat[bsel] = None
        lookup_block(p, bufs[bsel])
        r0 = p * B + base_b
        scat[bsel] = [
            pltpu.async_copy(
                bufs[bsel].at[cg],
                out_hbm.at[pl.ds(cg * 8, 8), pl.ds(r0, BPW)],
                ssem[bsel])
            for cg in range(D // 8)
        ]
    for s in scat:
        for cp in s:
            cp.wait()


def kernel(frame_index, table):
    fi_t = jnp.transpose(frame_index, (3, 1, 2, 0)).reshape(NFRAMES * B)
    tab_t = jnp.transpose(table).reshape(D * NROWS)
    out_t = _sc_lookup(fi_t, tab_t)
    return jnp.transpose(out_t).reshape(P * B, 1, 1, D)


# speedup vs baseline: 1.0293x; 1.0293x over previous
import functools

import jax
import jax.numpy as jnp
from jax import lax
from jax.experimental import pallas as pl
from jax.experimental.pallas import tpu as pltpu
from jax.experimental.pallas import tpu_sc as plsc

B = 16384
P = 10
NFRAMES = 20
D = 32
NROWS = 1000
OFFSET = 500
NC = 2
NS = 16
NW = NC * NS
BPW = B // NW

_mesh = plsc.VectorSubcoreMesh(core_axis_name="c", subcore_axis_name="s")


@functools.partial(
    pl.kernel,
    out_type=jax.ShapeDtypeStruct((D, P * B), jnp.float32),
    mesh=_mesh,
    compiler_params=pltpu.CompilerParams(
        needs_layout_passes=False, use_tc_tiling_on_sc=True),
    scratch_types=[
        pltpu.VMEM((NFRAMES * BPW,), jnp.int32),
        pltpu.VMEM((D * NROWS,), jnp.float32),
        pltpu.VMEM((P * BPW,), jnp.int32),
        pltpu.VMEM((D // 8, 8, BPW), jnp.float32),
        pltpu.VMEM((D // 8, 8, BPW), jnp.float32),
        pltpu.SemaphoreType.DMA,
        pltpu.SemaphoreType.DMA,
        pltpu.SemaphoreType.DMA,
    ],
)
def _sc_lookup(fi_hbm, table_hbm, out_hbm, fi_v, tab_v, idx_v,
               buf0, buf1, fsem, ssem0, ssem1):
    wid = lax.axis_index("s") * NC + lax.axis_index("c")
    base_b = wid * BPW

    tcopy = pltpu.async_copy(table_hbm, tab_v, fsem)
    fcopies = [
        pltpu.async_copy(fi_hbm.at[pl.ds(f * B + base_b, BPW)],
                         fi_v.at[pl.ds(f * BPW, BPW)], fsem)
        for f in range(NFRAMES)
    ]
    tcopy.wait()
    for cp in fcopies:
        cp.wait()

    @plsc.parallel_loop(0, P * (BPW // 16))
    def compute_idx(k):
        p = k // (BPW // 16)
        j = k - p * (BPW // 16)
        nxt = fi_v[pl.ds((p + 10) * BPW + j * 16, 16)]
        prv = fi_v[pl.ds(p * BPW + j * 16, 16)]
        idx_v[pl.ds(k * 16, 16)] = nxt - prv + OFFSET

    bufs = (buf0, buf1)
    ssem = (ssem0, ssem1)

    def lookup_block(p, buf):
        @plsc.parallel_loop(0, BPW // 16)
        def group(j0):
            idx = idx_v[pl.ds(p * BPW + j0 * 16, 16)]
            addrs = [idx + c * NROWS for c in range(D)]
            vals = [plsc.load_gather(tab_v, [a]) for a in addrs]
            for c in range(D):
                buf[c // 8, c % 8, pl.ds(j0 * 16, 16)] = vals[c]

    scat = [None, None]
    for p in range(P):
        bsel = p & 1
        if scat[bsel] is not None:
            for cp in scat[bsel]:
                cp.wait()
            scat[bsel] = None
        lookup_block(p, bufs[bsel])
        r0 = p * B + base_b
        scat[bsel] = [
            pltpu.async_copy(
                bufs[bsel].at[cg],
                out_hbm.at[pl.ds(cg * 8, 8), pl.ds(r0, BPW)],
                ssem[bsel])
            for cg in range(D // 8)
        ]
    for s in scat:
        for cp in s:
            cp.wait()


def kernel(frame_index, table):
    fi_t = jnp.transpose(frame_index, (3, 1, 2, 0)).reshape(NFRAMES * B)
    tab_t = jnp.transpose(table).reshape(D * NROWS)
    out_t = _sc_lookup(fi_t, tab_t)
    return jnp.transpose(out_t).reshape(P * B, 1, 1, D)
